# baseline (speedup 1.0000x reference)
import functools

import jax
import jax.numpy as jnp
from jax import lax
from jax.experimental import pallas as pl
from jax.experimental.pallas import tpu as pltpu
from jax.experimental.pallas import tpu_sc as plsc

NC = 2
NS = 16
NW = NC * NS
LANES = 16
D = 128
BM = 256


def _proj_body(x_ref, wt_ref, b_ref, o_ref):
    o_ref[...] = (
        jnp.dot(x_ref[...], wt_ref[...], preferred_element_type=jnp.float32)
        + b_ref[...]
    )


def _proj(x, wt, b2d):
    n = x.shape[0]
    return pl.pallas_call(
        _proj_body,
        grid=(n // BM,),
        in_specs=[
            pl.BlockSpec((BM, D), lambda i: (i, 0)),
            pl.BlockSpec((D, D), lambda i: (0, 0)),
            pl.BlockSpec((1, D), lambda i: (0, 0)),
        ],
        out_specs=pl.BlockSpec((BM, D), lambda i: (i, 0)),
        out_shape=jax.ShapeDtypeStruct((n, D), jnp.float32),
    )(x, wt, b2d)


def _combine_body(relu, s0_ref, s1_ref, c0_ref, c1_ref, xd_ref, wlt_ref,
                  bl_ref, wrt_ref, o_ref):
    s = s0_ref[0] + s1_ref[0]
    c = c0_ref[0] + c1_ref[0]
    mean = s / jnp.maximum(c, 1.0)
    out = (
        jnp.dot(mean, wlt_ref[...], preferred_element_type=jnp.float32)
        + bl_ref[...]
        + jnp.dot(xd_ref[...], wrt_ref[...], preferred_element_type=jnp.float32)
    )
    nrm = jnp.sqrt(jnp.sum(out * out, axis=1, keepdims=True))
    out = out / jnp.maximum(nrm, 1e-12)
    if relu:
        out = jnp.maximum(out, 0.0)
    o_ref[...] = out


def _combine(sums, cnts, x_dst, wlt, bl2d, wrt, relu, acc_rows, rows_per_pass):
    n = x_dst.shape[0]
    accb = acc_rows // BM
    rppb = rows_per_pass // BM

    def seg_map(i):
        return ((i // rppb) * accb + i % rppb, 0)

    return pl.pallas_call(
        functools.partial(_combine_body, relu),
        grid=(n // BM,),
        in_specs=[
            pl.BlockSpec((1, BM, D), lambda i: (0,) + seg_map(i)),
            pl.BlockSpec((1, BM, D), lambda i: (1,) + seg_map(i)),
            pl.BlockSpec((1, BM, 1), lambda i: (0,) + seg_map(i)),
            pl.BlockSpec((1, BM, 1), lambda i: (1,) + seg_map(i)),
            pl.BlockSpec((BM, D), lambda i: (i, 0)),
            pl.BlockSpec((D, D), lambda i: (0, 0)),
            pl.BlockSpec((1, D), lambda i: (0, 0)),
            pl.BlockSpec((D, D), lambda i: (0, 0)),
        ],
        out_specs=pl.BlockSpec((BM, D), lambda i: (i, 0)),
        out_shape=jax.ShapeDtypeStruct((n, D), jnp.float32),
    )(sums, sums, cnts, cnts, x_dst, wlt, bl2d, wrt)


def _scalar(v16):
    return jnp.reshape(lax.slice(v16, (0,), (1,)), ())


def _segsum_direct(acc_rows, rows_per_pass, e_pad, ch, with_cnt):
    ept = e_pad // NW
    n_chunk = ept // ch
    rpt = acc_rows // NS
    mesh = plsc.VectorSubcoreMesh(
        core_axis_name="c", subcore_axis_name="s", num_cores=NC,
        num_subcores=NS)
    sums_t = jax.ShapeDtypeStruct((NC, 1, acc_rows, D), jnp.float32)
    if with_cnt:
        out_type = [sums_t,
                    jax.ShapeDtypeStruct((NC * acc_rows,), jnp.float32)]
    else:
        out_type = sums_t

    @functools.partial(
        pl.kernel,
        mesh=mesh,
        out_type=out_type,
        scratch_types=[
            pltpu.VMEM((ept,), jnp.int32),
            pltpu.VMEM((ept,), jnp.int32),
            pltpu.VMEM((ch,), jnp.int32),
            pltpu.VMEM((2, ch, D), jnp.float32),
            pltpu.VMEM((ch,), jnp.float32),
            pltpu.VMEM_SHARED((acc_rows, D), jnp.float32),
            pltpu.VMEM_SHARED((acc_rows,), jnp.float32),
            pltpu.SemaphoreType.DMA,
            pltpu.SemaphoreType.DMA,
        ],
    )
    def kern(feat_hbm, src_hbm, dst_hbm, z2_hbm, z1_hbm, *out_and_scratch):
        if with_cnt:
            (sums_hbm, cnts_hbm, src_v, dst_v, lidx_v, rows_v, ones_v,
             acc_sh, cnt_sh, sem0, sem1) = out_and_scratch
        else:
            (sums_hbm, src_v, dst_v, lidx_v, rows_v, ones_v,
             acc_sh, cnt_sh, sem0, sem1) = out_and_scratch
        c = lax.axis_index("c")
        s = lax.axis_index("s")
        sems = (sem0, sem1)
        ebase = c * (NS * ept) + s * ept
        for j in range(ch // LANES):
            ones_v[pl.ds(j * LANES, LANES)] = jnp.ones((LANES,), jnp.float32)
        pltpu.sync_copy(src_hbm.at[pl.ds(ebase, ept)], src_v)
        pltpu.sync_copy(dst_hbm.at[pl.ds(ebase, ept)], dst_v)
        pltpu.sync_copy(z2_hbm.at[pl.ds(0, rpt), :],
                        acc_sh.at[pl.ds(s * rpt, rpt), :])

        if with_cnt:
            @pl.when(s == 0)
            def _zero_cnt():
                pltpu.sync_copy(z1_hbm.at[pl.ds(0, acc_rows)], cnt_sh)

        plsc.subcore_barrier()

        def fire(chunk, buf):
            pltpu.async_copy(feat_hbm.at[src_v.at[pl.ds(chunk * ch, ch)]],
                             rows_v.at[buf], sems[buf])

        def drain(buf):
            pltpu.make_async_copy(feat_hbm.at[src_v.at[pl.ds(0, ch)]],
                                  rows_v.at[buf], sems[buf]).wait()

        def scatter(chunk, buf):
            for j in range(ch // LANES):
                sl = pl.ds(j * LANES, LANES)
                lv = dst_v[pl.ds(chunk * ch + j * LANES, LANES)]
                ok = (lv >= 0) & (lv < rows_per_pass)
                lidx_v[sl] = jnp.where(ok, lv, rows_per_pass)
            pltpu.sync_copy(rows_v.at[buf], acc_sh.at[lidx_v], add=True)
            if with_cnt:
                pltpu.sync_copy(ones_v, cnt_sh.at[lidx_v], add=True)

        fire(0, 0)

        def pair_body(g, carry):
            fire(2 * g + 1, 1)
            drain(0)
            scatter(2 * g, 0)
            nxt = 2 * g + 2
            nxt = jnp.where(nxt >= n_chunk, 0, nxt)
            fire(nxt, 0)
            drain(1)
            scatter(2 * g + 1, 1)
            return carry

        lax.fori_loop(0, n_chunk // 2, pair_body, 0)
        drain(0)
        plsc.subcore_barrier()
        pltpu.sync_copy(acc_sh.at[pl.ds(s * rpt, rpt), :],
                        sums_hbm.at[c, 0, pl.ds(s * rpt, rpt), :])

        if with_cnt:
            @pl.when(s == 0)
            def _dump_cnt():
                pltpu.sync_copy(
                    cnt_sh, cnts_hbm.at[pl.ds(c * acc_rows, acc_rows)])

        plsc.subcore_barrier()

    return kern


def _segsum_compact(n_pass, acc_rows, rows_per_pass, e_pad, ch, sb, with_cnt):
    ept = e_pad // NW
    n_sb = ept // sb
    cap = ept + n_pass * ch + LANES
    rpt = acc_rows // NS
    mesh = plsc.VectorSubcoreMesh(
        core_axis_name="c", subcore_axis_name="s", num_cores=NC,
        num_subcores=NS)
    sums_t = jax.ShapeDtypeStruct((NC, n_pass, acc_rows, D), jnp.float32)
    if with_cnt:
        out_type = [
            sums_t,
            jax.ShapeDtypeStruct((NC * n_pass * acc_rows,), jnp.float32)]
    else:
        out_type = sums_t

    @functools.partial(
        pl.kernel,
        mesh=mesh,
        compiler_params=pltpu.CompilerParams(needs_layout_passes=False),
        out_type=out_type,
        scratch_types=[
            pltpu.VMEM((sb,), jnp.int32),
            pltpu.VMEM((sb,), jnp.int32),
            pltpu.VMEM((cap,), jnp.int32),
            pltpu.VMEM((cap,), jnp.int32),
            pltpu.VMEM((ch,), jnp.int32),
            pltpu.VMEM((2, ch, D), jnp.float32),
            pltpu.VMEM((ch,), jnp.float32),
            pltpu.VMEM_SHARED((acc_rows, D), jnp.float32),
            pltpu.VMEM_SHARED((acc_rows,), jnp.float32),
            pltpu.SemaphoreType.DMA,
            pltpu.SemaphoreType.DMA,
        ],
    )
    def kern(feat_hbm, src_hbm, dst_hbm, z2_hbm, z1_hbm, *out_and_scratch):
        if with_cnt:
            (sums_hbm, cnts_hbm, srcb_v, dstb_v, srcl_v, lidxl_v, lidx_v,
             rows_v, ones_v, acc_sh, cnt_sh, sem0, sem1) = out_and_scratch
        else:
            (sums_hbm, srcb_v, dstb_v, srcl_v, lidxl_v, lidx_v,
             rows_v, ones_v, acc_sh, cnt_sh, sem0, sem1) = out_and_scratch
        c = lax.axis_index("c")
        s = lax.axis_index("s")
        sems = (sem0, sem1)
        ebase = c * (NS * ept) + s * ept
        for j in range(ch // LANES):
            ones_v[pl.ds(j * LANES, LANES)] = jnp.ones((LANES,), jnp.float32)

        zero16 = jnp.zeros((LANES,), jnp.int32)
        dummy16 = jnp.full((LANES,), rows_per_pass, jnp.int32)

        def fill_body(g, carry):
            srcl_v[pl.ds(g * LANES, LANES)] = zero16
            lidxl_v[pl.ds(g * LANES, LANES)] = dummy16
            return carry

        lax.fori_loop(0, cap // LANES, fill_body, 0)

        def count_outer(t, counts):
            pltpu.sync_copy(dst_hbm.at[pl.ds(ebase + t * sb, sb)], dstb_v)

            def count_body(g, counts):
                d = dstb_v[pl.ds(g * LANES, LANES)]
                return tuple(
                    counts[p] + plsc.all_reduce_population_count(
                        (d >= p * rows_per_pass)
                        & (d < (p + 1) * rows_per_pass))
                    for p in range(n_pass))

            return lax.fori_loop(0, sb // LANES, count_body, counts)

        counts = lax.fori_loop(0, n_sb, count_outer,
                               tuple(zero16 for _ in range(n_pass)))
        n_p = [_scalar(counts[p]) for p in range(n_pass)]
        offs = [jnp.int32(0)]
        for p in range(n_pass):
            nxt = offs[p] + ((n_p[p] + (ch - 1)) // ch) * ch
            offs.append(nxt)

        def compact_outer(t, cur):
            pltpu.sync_copy(src_hbm.at[pl.ds(ebase + t * sb, sb)], srcb_v)
            pltpu.sync_copy(dst_hbm.at[pl.ds(ebase + t * sb, sb)], dstb_v)

            def compact_body(g, cur):
                d = dstb_v[pl.ds(g * LANES, LANES)]
                v = srcb_v[pl.ds(g * LANES, LANES)]
                out = []
                for p in range(n_pass):
                    lv = d - p * rows_per_pass
                    ok = (lv >= 0) & (lv < rows_per_pass)
                    plsc.store_compressed(srcl_v.at[pl.ds(cur[p], LANES)],
                                          v, mask=ok)
                    plsc.store_compressed(lidxl_v.at[pl.ds(cur[p], LANES)],
                                          lv, mask=ok)
                    out.append(
                        cur[p]
                        + _scalar(plsc.all_reduce_population_count(ok)))
                return tuple(out)

            return lax.fori_loop(0, sb // LANES, compact_body, cur)

        lax.fori_loop(0, n_sb, compact_outer, tuple(offs[:n_pass]))

        def fire(off, chunk, buf):
            pltpu.async_copy(
                feat_hbm.at[srcl_v.at[pl.ds(off + chunk * ch, ch)]],
                rows_v.at[buf], sems[buf])

        def drain(buf):
            pltpu.make_async_copy(feat_hbm.at[srcl_v.at[pl.ds(0, ch)]],
                                  rows_v.at[buf], sems[buf]).wait()

        def scatter(off, chunk, buf):
            for j in range(ch // LANES):
                sl = pl.ds(j * LANES, LANES)
                lidx_v[sl] = lidxl_v[pl.ds(off + chunk * ch + j * LANES,
                                           LANES)]
            pltpu.sync_copy(rows_v.at[buf], acc_sh.at[lidx_v], add=True)
            if with_cnt:
                pltpu.sync_copy(ones_v, cnt_sh.at[lidx_v], add=True)

        for p in range(n_pass):
            pltpu.sync_copy(z2_hbm.at[pl.ds(0, rpt), :],
                            acc_sh.at[pl.ds(s * rpt, rpt), :])

            if with_cnt:
                @pl.when(s == 0)
                def _zero_cnt():
                    pltpu.sync_copy(z1_hbm.at[pl.ds(0, acc_rows)], cnt_sh)

            plsc.subcore_barrier()

            off = offs[p]
            nch = (n_p[p] + (ch - 1)) // ch
            fire(off, 0, 0)

            def pair_body(g, carry):
                c1 = 2 * g + 1
                c1c = jnp.where(c1 >= nch, 0, c1)
                fire(off, c1c, 1)
                drain(0)
                scatter(off, 2 * g, 0)
                c2 = 2 * g + 2
                c2c = jnp.where(c2 >= nch, 0, c2)
                fire(off, c2c, 0)
                drain(1)

                @pl.when(c1 < nch)
                def _do():
                    scatter(off, c1, 1)

                return carry

            lax.fori_loop(0, (nch + 1) // 2, pair_body, 0)
            drain(0)
            plsc.subcore_barrier()
            pltpu.sync_copy(acc_sh.at[pl.ds(s * rpt, rpt), :],
                            sums_hbm.at[c, p, pl.ds(s * rpt, rpt), :])

            if with_cnt:
                @pl.when(s == 0)
                def _dump_cnt():
                    pltpu.sync_copy(
                        cnt_sh,
                        cnts_hbm.at[pl.ds((c * n_pass + p) * acc_rows,
                                          acc_rows)])

            plsc.subcore_barrier()

    return kern


def _dot_kernel(e_pad, ch):
    ept = e_pad // NW
    n_chunk = ept // ch
    mesh = plsc.VectorSubcoreMesh(
        core_axis_name="c", subcore_axis_name="s", num_cores=NC,
        num_subcores=NS)

    @functools.partial(
        pl.kernel,
        mesh=mesh,
        compiler_params=pltpu.CompilerParams(needs_layout_passes=False),
        out_type=jax.ShapeDtypeStruct((e_pad,), jnp.float32),
        scratch_types=[
            pltpu.VMEM((ept,), jnp.int32),
            pltpu.VMEM((ept,), jnp.int32),
            pltpu.VMEM((2, ch, D), jnp.float32),
            pltpu.VMEM((2, ch, D), jnp.float32),
            pltpu.VMEM((ch,), jnp.float32),
            pltpu.VMEM((LANES, LANES), jnp.float32),
            pltpu.SemaphoreType.DMA,
            pltpu.SemaphoreType.DMA,
            pltpu.SemaphoreType.DMA,
            pltpu.SemaphoreType.DMA,
        ],
    )
    def kern(a_hbm, b_hbm, ia_hbm, ib_hbm, out_hbm,
             ia_v, ib_v, a_v, b_v, o_v, m_v, sa0, sa1, sb0, sb1):
        c = lax.axis_index("c")
        s = lax.axis_index("s")
        sems_a = (sa0, sa1)
        sems_b = (sb0, sb1)
        ebase = c * (NS * ept) + s * ept
        pltpu.sync_copy(ia_hbm.at[pl.ds(ebase, ept)], ia_v)
        pltpu.sync_copy(ib_hbm.at[pl.ds(ebase, ept)], ib_v)

        def fire(chunk, buf):
            pltpu.async_copy(a_hbm.at[ia_v.at[pl.ds(chunk * ch, ch)]],
                             a_v.at[buf], sems_a[buf])
            pltpu.async_copy(b_hbm.at[ib_v.at[pl.ds(chunk * ch, ch)]],
                             b_v.at[buf], sems_b[buf])

        def drain(buf):
            pltpu.make_async_copy(a_hbm.at[ia_v.at[pl.ds(0, ch)]],
                                  a_v.at[buf], sems_a[buf]).wait()
            pltpu.make_async_copy(b_hbm.at[ib_v.at[pl.ds(0, ch)]],
                                  b_v.at[buf], sems_b[buf]).wait()

        def compute(chunk, buf):
            lane = lax.iota(jnp.int32, LANES)

            def group_body(g, carry2):
                r0 = g * LANES
                for k in range(LANES):
                    r = r0 + k
                    acc = (a_v[buf, r, pl.ds(0, LANES)]
                           * b_v[buf, r, pl.ds(0, LANES)])
                    for j in range(1, D // LANES):
                        sl = pl.ds(j * LANES, LANES)
                        acc = acc + a_v[buf, r, sl] * b_v[buf, r, sl]
                    m_v[k, pl.ds(0, LANES)] = acc
                tot = plsc.load_gather(m_v, [lane, jnp.zeros_like(lane)])
                for q in range(1, LANES):
                    tot = tot + plsc.load_gather(
                        m_v, [lane, jnp.full_like(lane, q)])
                o_v[pl.ds(r0, LANES)] = tot
                return carry2

            lax.fori_loop(0, ch // LANES, group_body, 0)
            pltpu.sync_copy(o_v, out_hbm.at[pl.ds(ebase + chunk * ch, ch)])

        fire(0, 0)

        def pair_body(g, carry):
            fire(2 * g + 1, 1)
            drain(0)
            compute(2 * g, 0)
            nxt = 2 * g + 2
            nxt = jnp.where(nxt >= n_chunk, 0, nxt)
            fire(nxt, 0)
            drain(1)
            compute(2 * g + 1, 1)
            return carry

        lax.fori_loop(0, n_chunk // 2, pair_body, 0)
        drain(0)

    return kern


def _pad_edges(ei, e_pad):
    e = ei.shape[1]
    pad = jnp.full((e_pad - e,), 0, jnp.int32)
    src = jnp.concatenate([ei[0], pad])
    dst = jnp.concatenate([ei[1], pad - 1])
    return src, dst


def _pad_rows(x, n_pad):
    return jnp.concatenate(
        [x, jnp.zeros((n_pad - x.shape[0], x.shape[1]), x.dtype)])


def kernel(x_track, x_playlist, edge_index_t2p, edge_index_p2t,
           edge_label_index, W_lin_t, b_lin_t, W_lin_p, b_lin_p,
           W1l_t2p, b1l_t2p, W1r_t2p, W1l_p2t, b1l_p2t, W1r_p2t,
           W2l_t2p, b2l_t2p, W2r_t2p, W2l_p2t, b2l_p2t, W2r_p2t):
    e = edge_index_t2p.shape[1]
    e_lbl = edge_label_index.shape[1]

    ch_p, ch_t, ch_l, sb_t = 80, 96, 128, 2000
    qp, ql = 2 * NW * ch_p, 2 * NW * ch_l
    ep_pad = ((e + qp - 1) // qp) * qp
    qt = NW * LANES
    et_pad = ((e + qt - 1) // qt) * qt
    el_pad = ((e_lbl + ql - 1) // ql) * ql

    p_pass, p_rows, p_acc, p_npad = 1, 10240, 10496, 10240
    t_pass, t_rows, t_acc, t_npad = 6, 8960, 9216, 53760

    zeros2d = jnp.zeros((p_acc // NS, D), jnp.float32)
    zeros1d = jnp.zeros((p_acc,), jnp.float32)

    xt = _pad_rows(x_track, t_npad)
    xp = _pad_rows(x_playlist, p_npad)

    src_t2p, dst_t2p = _pad_edges(edge_index_t2p, ep_pad)
    src_p2t, dst_p2t = _pad_edges(edge_index_p2t, et_pad)

    seg_t2p = _segsum_direct(p_acc, p_rows, ep_pad, ch_p, True)
    seg_p2t = _segsum_compact(t_pass, t_acc, t_rows, et_pad, ch_t, sb_t, True)
    seg_t2p_nc = _segsum_direct(p_acc, p_rows, ep_pad, ch_p, False)
    seg_p2t_nc = _segsum_compact(t_pass, t_acc, t_rows, et_pad, ch_t, sb_t,
                                 False)

    def seg_view(sums, cnts, n_pass, acc_rows):
        return (sums.reshape(NC, n_pass * acc_rows, D),
                cnts.reshape(NC, n_pass * acc_rows, 1))

    h_t = _proj(xt, W_lin_t.T, b_lin_t.reshape(1, D))
    h_p = _proj(xp, W_lin_p.T, b_lin_p.reshape(1, D))

    s_raw, c_raw = seg_t2p(h_t, src_t2p, dst_t2p, zeros2d, zeros1d)
    s_p, c_p = seg_view(s_raw, c_raw, p_pass, p_acc)
    p1 = _combine(s_p, c_p, h_p, W1l_t2p.T, b1l_t2p.reshape(1, D),
                  W1r_t2p.T, True, p_acc, p_rows)

    s_raw, c_raw = seg_p2t(h_p, src_p2t, dst_p2t, zeros2d, zeros1d)
    s_t, c_t = seg_view(s_raw, c_raw, t_pass, t_acc)
    t1 = _combine(s_t, c_t, h_t, W1l_p2t.T, b1l_p2t.reshape(1, D),
                  W1r_p2t.T, True, t_acc, t_rows)

    s_raw = seg_t2p_nc(t1, src_t2p, dst_t2p, zeros2d, zeros1d)
    s_p2 = s_raw.reshape(NC, p_pass * p_acc, D)
    p2 = _combine(s_p2, c_p, p1, W2l_t2p.T, b2l_t2p.reshape(1, D),
                  W2r_t2p.T, False, p_acc, p_rows)

    s_raw = seg_p2t_nc(p1, src_p2t, dst_p2t, zeros2d, zeros1d)
    s_t2 = s_raw.reshape(NC, t_pass * t_acc, D)
    t2 = _combine(s_t2, c_t, t1, W2l_p2t.T, b2l_p2t.reshape(1, D),
                  W2r_p2t.T, False, t_acc, t_rows)

    pad = jnp.zeros((el_pad - e_lbl,), jnp.int32)
    ia = jnp.concatenate([edge_label_index[0], pad])
    ib = jnp.concatenate([edge_label_index[1], pad])
    pred = _dot_kernel(el_pad, ch_l)(t2, p2, ia, ib)
    return pred[:e_lbl]


# speedup vs baseline: 1.1522x; 1.1522x over previous
import functools

import jax
import jax.numpy as jnp
from jax import lax
from jax.experimental import pallas as pl
from jax.experimental.pallas import tpu as pltpu
from jax.experimental.pallas import tpu_sc as plsc

NC = 2
NS = 16
NW = NC * NS
LANES = 16
D = 128
BM = 256


def _proj_body(x_ref, wt_ref, b_ref, o_ref):
    o_ref[...] = (
        jnp.dot(x_ref[...], wt_ref[...], preferred_element_type=jnp.float32)
        + b_ref[...]
    )


def _proj(x, wt, b2d):
    n = x.shape[0]
    return pl.pallas_call(
        _proj_body,
        grid=(n // BM,),
        in_specs=[
            pl.BlockSpec((BM, D), lambda i: (i, 0)),
            pl.BlockSpec((D, D), lambda i: (0, 0)),
            pl.BlockSpec((1, D), lambda i: (0, 0)),
        ],
        out_specs=pl.BlockSpec((BM, D), lambda i: (i, 0)),
        out_shape=jax.ShapeDtypeStruct((n, D), jnp.float32),
    )(x, wt, b2d)


def _combine_body(relu, s0_ref, s1_ref, c0_ref, c1_ref, xd_ref, wlt_ref,
                  bl_ref, wrt_ref, o_ref):
    s = s0_ref[0] + s1_ref[0]
    c = c0_ref[0] + c1_ref[0]
    mean = s / jnp.maximum(c, 1.0)
    out = (
        jnp.dot(mean, wlt_ref[...], preferred_element_type=jnp.float32)
        + bl_ref[...]
        + jnp.dot(xd_ref[...], wrt_ref[...], preferred_element_type=jnp.float32)
    )
    nrm = jnp.sqrt(jnp.sum(out * out, axis=1, keepdims=True))
    out = out / jnp.maximum(nrm, 1e-12)
    if relu:
        out = jnp.maximum(out, 0.0)
    o_ref[...] = out


def _combine(sums, cnts, x_dst, wlt, bl2d, wrt, relu, acc_rows, rows_per_pass):
    n = x_dst.shape[0]
    accb = acc_rows // BM
    rppb = rows_per_pass // BM

    def seg_map(i):
        return ((i // rppb) * accb + i % rppb, 0)

    return pl.pallas_call(
        functools.partial(_combine_body, relu),
        grid=(n // BM,),
        in_specs=[
            pl.BlockSpec((1, BM, D), lambda i: (0,) + seg_map(i)),
            pl.BlockSpec((1, BM, D), lambda i: (1,) + seg_map(i)),
            pl.BlockSpec((1, BM, 1), lambda i: (0,) + seg_map(i)),
            pl.BlockSpec((1, BM, 1), lambda i: (1,) + seg_map(i)),
            pl.BlockSpec((BM, D), lambda i: (i, 0)),
            pl.BlockSpec((D, D), lambda i: (0, 0)),
            pl.BlockSpec((1, D), lambda i: (0, 0)),
            pl.BlockSpec((D, D), lambda i: (0, 0)),
        ],
        out_specs=pl.BlockSpec((BM, D), lambda i: (i, 0)),
        out_shape=jax.ShapeDtypeStruct((n, D), jnp.float32),
    )(sums, sums, cnts, cnts, x_dst, wlt, bl2d, wrt)


def _scalar(v16):
    return jnp.reshape(lax.slice(v16, (0,), (1,)), ())


def _segsum_direct(acc_rows, rows_per_pass, e_pad, ch, with_cnt):
    ept = e_pad // NW
    n_chunk = ept // ch
    rpt = acc_rows // NS
    mesh = plsc.VectorSubcoreMesh(
        core_axis_name="c", subcore_axis_name="s", num_cores=NC,
        num_subcores=NS)
    sums_t = jax.ShapeDtypeStruct((NC, 1, acc_rows, D), jnp.float32)
    if with_cnt:
        out_type = [sums_t,
                    jax.ShapeDtypeStruct((NC * acc_rows,), jnp.float32)]
    else:
        out_type = sums_t

    @functools.partial(
        pl.kernel,
        mesh=mesh,
        out_type=out_type,
        scratch_types=[
            pltpu.VMEM((ept,), jnp.int32),
            pltpu.VMEM((ept,), jnp.int32),
            pltpu.VMEM((ch,), jnp.int32),
            pltpu.VMEM((2, ch, D), jnp.float32),
            pltpu.VMEM((ch,), jnp.float32),
            pltpu.VMEM_SHARED((acc_rows, D), jnp.float32),
            pltpu.VMEM_SHARED((acc_rows,), jnp.float32),
            pltpu.SemaphoreType.DMA,
            pltpu.SemaphoreType.DMA,
        ],
    )
    def kern(feat_hbm, src_hbm, dst_hbm, z2_hbm, z1_hbm, *out_and_scratch):
        if with_cnt:
            (sums_hbm, cnts_hbm, src_v, dst_v, lidx_v, rows_v, ones_v,
             acc_sh, cnt_sh, sem0, sem1) = out_and_scratch
        else:
            (sums_hbm, src_v, dst_v, lidx_v, rows_v, ones_v,
             acc_sh, cnt_sh, sem0, sem1) = out_and_scratch
        c = lax.axis_index("c")
        s = lax.axis_index("s")
        sems = (sem0, sem1)
        ebase = c * (NS * ept) + s * ept
        for j in range(ch // LANES):
            ones_v[pl.ds(j * LANES, LANES)] = jnp.ones((LANES,), jnp.float32)
        pltpu.sync_copy(src_hbm.at[pl.ds(ebase, ept)], src_v)
        pltpu.sync_copy(dst_hbm.at[pl.ds(ebase, ept)], dst_v)
        pltpu.sync_copy(z2_hbm.at[pl.ds(0, rpt), :],
                        acc_sh.at[pl.ds(s * rpt, rpt), :])

        if with_cnt:
            @pl.when(s == 0)
            def _zero_cnt():
                pltpu.sync_copy(z1_hbm.at[pl.ds(0, acc_rows)], cnt_sh)

        plsc.subcore_barrier()

        def fire(chunk, buf):
            pltpu.async_copy(feat_hbm.at[src_v.at[pl.ds(chunk * ch, ch)]],
                             rows_v.at[buf], sems[buf])

        def drain(buf):
            pltpu.make_async_copy(feat_hbm.at[src_v.at[pl.ds(0, ch)]],
                                  rows_v.at[buf], sems[buf]).wait()

        def scatter(chunk, buf):
            for j in range(ch // LANES):
                sl = pl.ds(j * LANES, LANES)
                lv = dst_v[pl.ds(chunk * ch + j * LANES, LANES)]
                ok = (lv >= 0) & (lv < rows_per_pass)
                lidx_v[sl] = jnp.where(ok, lv, rows_per_pass)
            pltpu.sync_copy(rows_v.at[buf], acc_sh.at[lidx_v], add=True)
            if with_cnt:
                pltpu.sync_copy(ones_v, cnt_sh.at[lidx_v], add=True)

        fire(0, 0)

        def pair_body(g, carry):
            fire(2 * g + 1, 1)
            drain(0)
            scatter(2 * g, 0)
            nxt = 2 * g + 2
            nxt = jnp.where(nxt >= n_chunk, 0, nxt)
            fire(nxt, 0)
            drain(1)
            scatter(2 * g + 1, 1)
            return carry

        lax.fori_loop(0, n_chunk // 2, pair_body, 0)
        drain(0)
        plsc.subcore_barrier()
        pltpu.sync_copy(acc_sh.at[pl.ds(s * rpt, rpt), :],
                        sums_hbm.at[c, 0, pl.ds(s * rpt, rpt), :])

        if with_cnt:
            @pl.when(s == 0)
            def _dump_cnt():
                pltpu.sync_copy(
                    cnt_sh, cnts_hbm.at[pl.ds(c * acc_rows, acc_rows)])

        plsc.subcore_barrier()

    return kern


def _segsum_compact(n_pass, acc_rows, rows_per_pass, e_pad, ch, sb, with_cnt):
    ept = e_pad // NW
    n_sb = ept // sb
    cap = ept + n_pass * ch + LANES
    rpt = acc_rows // NS
    mesh = plsc.VectorSubcoreMesh(
        core_axis_name="c", subcore_axis_name="s", num_cores=NC,
        num_subcores=NS)
    sums_t = jax.ShapeDtypeStruct((NC, n_pass, acc_rows, D), jnp.float32)
    if with_cnt:
        out_type = [
            sums_t,
            jax.ShapeDtypeStruct((NC * n_pass * acc_rows,), jnp.float32)]
    else:
        out_type = sums_t

    @functools.partial(
        pl.kernel,
        mesh=mesh,
        compiler_params=pltpu.CompilerParams(needs_layout_passes=False),
        out_type=out_type,
        scratch_types=[
            pltpu.VMEM((sb,), jnp.int32),
            pltpu.VMEM((sb,), jnp.int32),
            pltpu.VMEM((cap,), jnp.int32),
            pltpu.VMEM((cap,), jnp.int32),
            pltpu.VMEM((ch,), jnp.int32),
            pltpu.VMEM((2, ch, D), jnp.float32),
            pltpu.VMEM((ch,), jnp.float32),
            pltpu.VMEM_SHARED((acc_rows, D), jnp.float32),
            pltpu.VMEM_SHARED((acc_rows,), jnp.float32),
            pltpu.SemaphoreType.DMA,
            pltpu.SemaphoreType.DMA,
        ],
    )
    def kern(feat_hbm, src_hbm, dst_hbm, z2_hbm, z1_hbm, *out_and_scratch):
        if with_cnt:
            (sums_hbm, cnts_hbm, srcb_v, dstb_v, srcl_v, lidxl_v, lidx_v,
             rows_v, ones_v, acc_sh, cnt_sh, sem0, sem1) = out_and_scratch
        else:
            (sums_hbm, srcb_v, dstb_v, srcl_v, lidxl_v, lidx_v,
             rows_v, ones_v, acc_sh, cnt_sh, sem0, sem1) = out_and_scratch
        c = lax.axis_index("c")
        s = lax.axis_index("s")
        sems = (sem0, sem1)
        ebase = c * (NS * ept) + s * ept
        for j in range(ch // LANES):
            ones_v[pl.ds(j * LANES, LANES)] = jnp.ones((LANES,), jnp.float32)

        zero16 = jnp.zeros((LANES,), jnp.int32)
        dummy16 = jnp.full((LANES,), rows_per_pass, jnp.int32)

        def fill_body(g, carry):
            srcl_v[pl.ds(g * LANES, LANES)] = zero16
            lidxl_v[pl.ds(g * LANES, LANES)] = dummy16
            return carry

        lax.fori_loop(0, cap // LANES, fill_body, 0)

        def count_outer(t, counts):
            pltpu.sync_copy(dst_hbm.at[pl.ds(ebase + t * sb, sb)], dstb_v)

            def count_body(g, counts):
                d = dstb_v[pl.ds(g * LANES, LANES)]
                return tuple(
                    counts[p] + plsc.all_reduce_population_count(
                        (d >= p * rows_per_pass)
                        & (d < (p + 1) * rows_per_pass))
                    for p in range(n_pass))

            return lax.fori_loop(0, sb // LANES, count_body, counts)

        counts = lax.fori_loop(0, n_sb, count_outer,
                               tuple(zero16 for _ in range(n_pass)))
        n_p = [_scalar(counts[p]) for p in range(n_pass)]
        offs = [jnp.int32(0)]
        for p in range(n_pass):
            nxt = offs[p] + ((n_p[p] + (ch - 1)) // ch) * ch
            offs.append(nxt)

        def compact_outer(t, cur):
            pltpu.sync_copy(src_hbm.at[pl.ds(ebase + t * sb, sb)], srcb_v)
            pltpu.sync_copy(dst_hbm.at[pl.ds(ebase + t * sb, sb)], dstb_v)

            def compact_body(g, cur):
                d = dstb_v[pl.ds(g * LANES, LANES)]
                v = srcb_v[pl.ds(g * LANES, LANES)]
                out = []
                for p in range(n_pass):
                    lv = d - p * rows_per_pass
                    ok = (lv >= 0) & (lv < rows_per_pass)
                    plsc.store_compressed(srcl_v.at[pl.ds(cur[p], LANES)],
                                          v, mask=ok)
                    plsc.store_compressed(lidxl_v.at[pl.ds(cur[p], LANES)],
                                          lv, mask=ok)
                    out.append(
                        cur[p]
                        + _scalar(plsc.all_reduce_population_count(ok)))
                return tuple(out)

            return lax.fori_loop(0, sb // LANES, compact_body, cur)

        lax.fori_loop(0, n_sb, compact_outer, tuple(offs[:n_pass]))

        def fire(off, chunk, buf):
            pltpu.async_copy(
                feat_hbm.at[srcl_v.at[pl.ds(off + chunk * ch, ch)]],
                rows_v.at[buf], sems[buf])

        def drain(buf):
            pltpu.make_async_copy(feat_hbm.at[srcl_v.at[pl.ds(0, ch)]],
                                  rows_v.at[buf], sems[buf]).wait()

        def scatter(off, chunk, buf):
            for j in range(ch // LANES):
                sl = pl.ds(j * LANES, LANES)
                lidx_v[sl] = lidxl_v[pl.ds(off + chunk * ch + j * LANES,
                                           LANES)]
            pltpu.sync_copy(rows_v.at[buf], acc_sh.at[lidx_v], add=True)
            if with_cnt:
                pltpu.sync_copy(ones_v, cnt_sh.at[lidx_v], add=True)

        for p in range(n_pass):
            pltpu.sync_copy(z2_hbm.at[pl.ds(0, rpt), :],
                            acc_sh.at[pl.ds(s * rpt, rpt), :])

            if with_cnt:
                @pl.when(s == 0)
                def _zero_cnt():
                    pltpu.sync_copy(z1_hbm.at[pl.ds(0, acc_rows)], cnt_sh)

            plsc.subcore_barrier()

            off = offs[p]
            nch = (n_p[p] + (ch - 1)) // ch
            fire(off, 0, 0)

            def pair_body(g, carry):
                c1 = 2 * g + 1
                c1c = jnp.where(c1 >= nch, 0, c1)
                fire(off, c1c, 1)
                drain(0)
                scatter(off, 2 * g, 0)
                c2 = 2 * g + 2
                c2c = jnp.where(c2 >= nch, 0, c2)
                fire(off, c2c, 0)
                drain(1)

                @pl.when(c1 < nch)
                def _do():
                    scatter(off, c1, 1)

                return carry

            lax.fori_loop(0, (nch + 1) // 2, pair_body, 0)
            drain(0)
            plsc.subcore_barrier()
            pltpu.sync_copy(acc_sh.at[pl.ds(s * rpt, rpt), :],
                            sums_hbm.at[c, p, pl.ds(s * rpt, rpt), :])

            if with_cnt:
                @pl.when(s == 0)
                def _dump_cnt():
                    pltpu.sync_copy(
                        cnt_sh,
                        cnts_hbm.at[pl.ds((c * n_pass + p) * acc_rows,
                                          acc_rows)])

            plsc.subcore_barrier()

    return kern


def _dot_kernel(e_pad, ch):
    ept = e_pad // NW
    n_chunk = ept // ch
    mesh = plsc.VectorSubcoreMesh(
        core_axis_name="c", subcore_axis_name="s", num_cores=NC,
        num_subcores=NS)

    @functools.partial(
        pl.kernel,
        mesh=mesh,
        compiler_params=pltpu.CompilerParams(needs_layout_passes=False),
        out_type=jax.ShapeDtypeStruct((e_pad,), jnp.float32),
        scratch_types=[
            pltpu.VMEM((ept,), jnp.int32),
            pltpu.VMEM((ept,), jnp.int32),
            pltpu.VMEM((2, ch, D), jnp.float32),
            pltpu.VMEM((2, ch, D), jnp.float32),
            pltpu.VMEM((ch,), jnp.float32),
            pltpu.VMEM((LANES, LANES), jnp.float32),
            pltpu.SemaphoreType.DMA,
            pltpu.SemaphoreType.DMA,
            pltpu.SemaphoreType.DMA,
            pltpu.SemaphoreType.DMA,
        ],
    )
    def kern(a_hbm, b_hbm, ia_hbm, ib_hbm, out_hbm,
             ia_v, ib_v, a_v, b_v, o_v, m_v, sa0, sa1, sb0, sb1):
        c = lax.axis_index("c")
        s = lax.axis_index("s")
        sems_a = (sa0, sa1)
        sems_b = (sb0, sb1)
        ebase = c * (NS * ept) + s * ept
        pltpu.sync_copy(ia_hbm.at[pl.ds(ebase, ept)], ia_v)
        pltpu.sync_copy(ib_hbm.at[pl.ds(ebase, ept)], ib_v)

        def fire(chunk, buf):
            pltpu.async_copy(a_hbm.at[ia_v.at[pl.ds(chunk * ch, ch)]],
                             a_v.at[buf], sems_a[buf])
            pltpu.async_copy(b_hbm.at[ib_v.at[pl.ds(chunk * ch, ch)]],
                             b_v.at[buf], sems_b[buf])

        def drain(buf):
            pltpu.make_async_copy(a_hbm.at[ia_v.at[pl.ds(0, ch)]],
                                  a_v.at[buf], sems_a[buf]).wait()
            pltpu.make_async_copy(b_hbm.at[ib_v.at[pl.ds(0, ch)]],
                                  b_v.at[buf], sems_b[buf]).wait()

        def compute(chunk, buf):
            lane = lax.iota(jnp.int32, LANES)

            def group_body(g, carry2):
                r0 = g * LANES
                for k in range(LANES):
                    r = r0 + k
                    acc = (a_v[buf, r, pl.ds(0, LANES)]
                           * b_v[buf, r, pl.ds(0, LANES)])
                    for j in range(1, D // LANES):
                        sl = pl.ds(j * LANES, LANES)
                        acc = acc + a_v[buf, r, sl] * b_v[buf, r, sl]
                    m_v[k, pl.ds(0, LANES)] = acc
                tot = plsc.load_gather(m_v, [lane, jnp.zeros_like(lane)])
                for q in range(1, LANES):
                    tot = tot + plsc.load_gather(
                        m_v, [lane, jnp.full_like(lane, q)])
                o_v[pl.ds(r0, LANES)] = tot
                return carry2

            lax.fori_loop(0, ch // LANES, group_body, 0)
            pltpu.sync_copy(o_v, out_hbm.at[pl.ds(ebase + chunk * ch, ch)])

        fire(0, 0)

        def pair_body(g, carry):
            fire(2 * g + 1, 1)
            drain(0)
            compute(2 * g, 0)
            nxt = 2 * g + 2
            nxt = jnp.where(nxt >= n_chunk, 0, nxt)
            fire(nxt, 0)
            drain(1)
            compute(2 * g + 1, 1)
            return carry

        lax.fori_loop(0, n_chunk // 2, pair_body, 0)
        drain(0)

    return kern


def _pad_edges(ei, e_pad):
    e = ei.shape[1]
    pad = jnp.full((e_pad - e,), 0, jnp.int32)
    src = jnp.concatenate([ei[0], pad])
    dst = jnp.concatenate([ei[1], pad - 1])
    return src, dst


def _pad_rows(x, n_pad):
    return jnp.concatenate(
        [x, jnp.zeros((n_pad - x.shape[0], x.shape[1]), x.dtype)])


def kernel(x_track, x_playlist, edge_index_t2p, edge_index_p2t,
           edge_label_index, W_lin_t, b_lin_t, W_lin_p, b_lin_p,
           W1l_t2p, b1l_t2p, W1r_t2p, W1l_p2t, b1l_p2t, W1r_p2t,
           W2l_t2p, b2l_t2p, W2r_t2p, W2l_p2t, b2l_p2t, W2r_p2t):
    e = edge_index_t2p.shape[1]
    e_lbl = edge_label_index.shape[1]

    ch_p, ch_t, ch_l, sb_t = 80, 48, 128, 2000
    qp, ql = 2 * NW * ch_p, 2 * NW * ch_l
    ep_pad = ((e + qp - 1) // qp) * qp
    qt = NW * LANES
    et_pad = ((e + qt - 1) // qt) * qt
    el_pad = ((e_lbl + ql - 1) // ql) * ql

    p_pass, p_rows, p_acc, p_npad = 1, 10240, 10496, 10240
    t_pass, t_rows, t_acc, t_npad = 6, 8960, 9216, 53760

    zeros2d = jnp.zeros((p_acc // NS, D), jnp.float32)
    zeros1d = jnp.zeros((p_acc,), jnp.float32)

    xt = _pad_rows(x_track, t_npad)
    xp = _pad_rows(x_playlist, p_npad)

    src_t2p, dst_t2p = _pad_edges(edge_index_t2p, ep_pad)
    src_p2t, dst_p2t = _pad_edges(edge_index_p2t, et_pad)

    seg_t2p = _segsum_direct(p_acc, p_rows, ep_pad, ch_p, True)
    seg_p2t = _segsum_compact(t_pass, t_acc, t_rows, et_pad, ch_t, sb_t, True)
    seg_t2p_nc = _segsum_direct(p_acc, p_rows, ep_pad, ch_p, False)
    seg_p2t_nc = _segsum_compact(t_pass, t_acc, t_rows, et_pad, ch_t, sb_t,
                                 False)

    def seg_view(sums, cnts, n_pass, acc_rows):
        return (sums.reshape(NC, n_pass * acc_rows, D),
                cnts.reshape(NC, n_pass * acc_rows, 1))

    h_t = _proj(xt, W_lin_t.T, b_lin_t.reshape(1, D))
    h_p = _proj(xp, W_lin_p.T, b_lin_p.reshape(1, D))

    s_raw, c_raw = seg_t2p(h_t, src_t2p, dst_t2p, zeros2d, zeros1d)
    s_p, c_p = seg_view(s_raw, c_raw, p_pass, p_acc)
    p1 = _combine(s_p, c_p, h_p, W1l_t2p.T, b1l_t2p.reshape(1, D),
                  W1r_t2p.T, True, p_acc, p_rows)

    s_raw, c_raw = seg_p2t(h_p, src_p2t, dst_p2t, zeros2d, zeros1d)
    s_t, c_t = seg_view(s_raw, c_raw, t_pass, t_acc)
    t1 = _combine(s_t, c_t, h_t, W1l_p2t.T, b1l_p2t.reshape(1, D),
                  W1r_p2t.T, True, t_acc, t_rows)

    s_raw = seg_t2p_nc(t1, src_t2p, dst_t2p, zeros2d, zeros1d)
    s_p2 = s_raw.reshape(NC, p_pass * p_acc, D)
    p2 = _combine(s_p2, c_p, p1, W2l_t2p.T, b2l_t2p.reshape(1, D),
                  W2r_t2p.T, False, p_acc, p_rows)

    s_raw = seg_p2t_nc(p1, src_p2t, dst_p2t, zeros2d, zeros1d)
    s_t2 = s_raw.reshape(NC, t_pass * t_acc, D)
    t2 = _combine(s_t2, c_t, t1, W2l_p2t.T, b2l_p2t.reshape(1, D),
                  W2r_p2t.T, False, t_acc, t_rows)

    pad = jnp.zeros((el_pad - e_lbl,), jnp.int32)
    ia = jnp.concatenate([edge_label_index[0], pad])
    ib = jnp.concatenate([edge_label_index[1], pad])
    pred = _dot_kernel(el_pad, ch_l)(t2, p2, ia, ib)
    return pred[:e_lbl]
